# Initial kernel scaffold; baseline (speedup 1.0000x reference)
import jax
import jax.numpy as jnp
from jax.experimental import pallas as pl


def kernel(mem_scene_keys, mem_path_candidates, mem_rewards, counter, scene_keys, path_candidates, rewards):
    raise NotImplementedError("write your pallas kernel here")


# speedup vs baseline: 2.1376x; 2.1376x over previous
import functools

import jax
import jax.numpy as jnp
from jax import lax
from jax.experimental import pallas as pl
from jax.experimental.pallas import tpu as pltpu
from jax.experimental.pallas import tpu_sc as plsc

CAP = 1_000_000
BATCH = 16384
ORDER = 16
NC = 2
NS = 16
NT = NC * NS
PER = BATCH // NT
L = 16


def _scatter_body(cnt_hbm, sk_hbm, pc_hbm, rw_hbm, mem_sk, mem_pc, mem_rw,
                  cnt_out, rwa, sko, pco, cntv, da2d, sem):
    wid = lax.axis_index("s") * NC + lax.axis_index("c")
    own = wid * PER

    in_copies = [
        pltpu.async_copy(rw_hbm, rwa, sem),
        pltpu.async_copy(sk_hbm.at[pl.ds(own, PER)], sko, sem),
        pltpu.async_copy(pc_hbm.at[pl.ds(own, PER)], pco, sem),
        pltpu.async_copy(cnt_hbm, cntv, sem),
    ]
    for c in in_copies:
        c.wait()

    ctr = cntv[...][0]
    one = jnp.full((L,), 1, jnp.int32)
    zero = jnp.full((L,), 0, jnp.int32)

    def count_block(b, acc):
        off = b * (8 * L)
        for k in range(8):
            v = rwa[pl.ds(off + k * L, L)]
            acc = acc + jnp.sum(jnp.where(v > 0.0, one, zero))
        return acc

    base = lax.fori_loop(0, wid * (PER // (8 * L)), count_block,
                         jnp.int32(0))

    run = base
    for j in range(PER // L):
        v = rwa[pl.ds(own + j * L, L)]
        m = v > 0.0
        mi = jnp.where(m, one, zero)
        excl = plsc.cumsum(mi) - mi
        dest = excl + (ctr + run)
        dest = jnp.where(dest >= CAP, dest - CAP, dest)
        da2d[j // 8, pl.ds((j % 8) * L, L)] = jnp.where(m, dest, -1)
        run = run + jnp.sum(mi)

    out_copies = []
    for q in range(4):
        idx = plsc.Indices(da2d.at[q], ignored_value=-1)
        out_copies.append(
            pltpu.async_copy(sko.at[pl.ds(q * 128, 128)], mem_sk.at[idx], sem))
        out_copies.append(
            pltpu.async_copy(pco.at[pl.ds(q * 128, 128)], mem_pc.at[idx], sem))
        out_copies.append(
            pltpu.async_copy(rwa.at[pl.ds(own + q * 128, 128)],
                             mem_rw.at[idx], sem))
    for c in out_copies:
        c.wait()

    @pl.when(wid == NT - 1)
    def _():
        cntv[...] = jnp.broadcast_to(ctr + run, (L,))
        pltpu.sync_copy(cntv, cnt_out)


_scatter_kernel = functools.partial(
    pl.kernel,
    out_type=jax.ShapeDtypeStruct((L,), jnp.int32),
    mesh=plsc.VectorSubcoreMesh(core_axis_name="c", subcore_axis_name="s"),
    compiler_params=pltpu.CompilerParams(use_tc_tiling_on_sc=False,
                                         needs_layout_passes=False),
    scratch_types=[
        pltpu.VMEM((BATCH,), jnp.float32),
        pltpu.VMEM((PER,), jnp.int32),
        pltpu.VMEM((PER, ORDER), jnp.int32),
        pltpu.VMEM((L,), jnp.int32),
        pltpu.VMEM((4, 128), jnp.int32),
        pltpu.SemaphoreType.DMA,
    ],
)(_scatter_body)


def kernel(mem_scene_keys, mem_path_candidates, mem_rewards, counter,
           scene_keys, path_candidates, rewards):
    cnt_b = jnp.broadcast_to(counter.astype(jnp.int32), (L,))
    sk_ref = jax.new_ref(mem_scene_keys)
    pc_ref = jax.new_ref(mem_path_candidates)
    rw_ref = jax.new_ref(mem_rewards)
    cnt_out = _scatter_kernel(cnt_b, scene_keys, path_candidates, rewards,
                              sk_ref, pc_ref, rw_ref)
    return (jax.freeze(sk_ref), jax.freeze(pc_ref), jax.freeze(rw_ref),
            cnt_out[0])
